# baseline (speedup 1.0000x reference)
import functools

import jax
import jax.numpy as jnp
from jax import lax
from jax.experimental import pallas as pl
from jax.experimental.pallas import tpu as pltpu
from jax.experimental.pallas import tpu_sc as plsc

NNODES = 10000
NFEAT = 128
ODIM = 128
NEDGES = 320000
T = 10

NC = 2
NS = 16
NW = NC * NS
EPT = NEDGES // NW
BLK = 40
NBLK = EPT // BLK
RPT = 624
TAIL = NNODES - NS * RPT

_SC_MESH = plsc.VectorSubcoreMesh(core_axis_name="c", subcore_axis_name="s")


@functools.partial(
    pl.kernel,
    out_type=jax.ShapeDtypeStruct((NC, NNODES, ODIM), jnp.float32),
    mesh=_SC_MESH,
    scratch_types=[
        pltpu.VMEM((EPT,), jnp.int32),
        pltpu.VMEM((EPT // 80, 80), jnp.int32),
        pltpu.VMEM((4 * BLK, ODIM), jnp.float32),
        pltpu.VMEM_SHARED((NNODES, ODIM), jnp.float32),
        pltpu.SemaphoreType.DMA,
        pltpu.SemaphoreType.DMA,
        pltpu.SemaphoreType.DMA,
        pltpu.SemaphoreType.DMA,
        pltpu.SemaphoreType.DMA,
        pltpu.SemaphoreType.DMA,
    ],
)
def _sc_spmm(p_hbm, src_hbm, dst_hbm, zeros_hbm, out_hbm,
             src_v, dst_v, rows_v, acc_sh, sem0, sem1, sem2, sem3,
             isemA, isemB):
    c = lax.axis_index("c")
    s = lax.axis_index("s")
    wid = s * NC + c
    pltpu.async_copy(src_hbm.at[wid], src_v, isemA)
    pltpu.async_copy(dst_hbm.at[wid], dst_v, isemB)
    pltpu.make_async_copy(src_hbm.at[wid], src_v, isemA).wait()

    sems = (sem0, sem1, sem2, sem3)

    def gather(j, b, sem):
        jc = jnp.minimum(j, NBLK - 1)
        return pltpu.async_copy(
            p_hbm.at[src_v.at[pl.ds(jc * BLK, BLK)]],
            rows_v.at[pl.ds(b * BLK, BLK)], sem)

    def gwait(b, sem):
        pltpu.make_async_copy(
            p_hbm.at[src_v.at[pl.ds(0, BLK)]],
            rows_v.at[pl.ds(b * BLK, BLK)], sem).wait()

    def scatter_pair(p, half):
        off = half * 2 * BLK
        pltpu.sync_copy(rows_v.at[pl.ds(off, 2 * BLK)],
                        acc_sh.at[dst_v.at[p]], add=True)

    gather(0, 0, sems[0])
    gather(1, 1, sems[1])
    gather(2, 2, sems[2])
    gather(3, 3, sems[3])
    pltpu.sync_copy(zeros_hbm, acc_sh.at[pl.ds(s * RPT, RPT)])

    @pl.when(s == 0)
    def _():
        pltpu.sync_copy(zeros_hbm.at[pl.ds(0, TAIL)],
                        acc_sh.at[pl.ds(NS * RPT, TAIL)])

    pltpu.make_async_copy(dst_hbm.at[wid], dst_v, isemB).wait()
    plsc.subcore_barrier()

    def body(i, carry):
        p = 2 * i
        j = 2 * p
        gwait(0, sems[0])
        gwait(1, sems[1])
        scatter_pair(p, 0)
        gather(j + 4, 0, sems[0])
        gather(j + 5, 1, sems[1])
        gwait(2, sems[2])
        gwait(3, sems[3])
        scatter_pair(p + 1, 1)
        gather(j + 6, 2, sems[2])
        gather(j + 7, 3, sems[3])
        return carry

    lax.fori_loop(0, (NBLK // 2) // 2, body, 0)
    gwait(0, sems[0])
    gwait(1, sems[1])
    scatter_pair((NBLK // 2) - 1, 0)
    gwait(2, sems[2])
    gwait(3, sems[3])
    plsc.subcore_barrier()

    pltpu.sync_copy(acc_sh.at[pl.ds(s * RPT, RPT)],
                    out_hbm.at[c, pl.ds(s * RPT, RPT)])

    @pl.when(s == 0)
    def _():
        pltpu.sync_copy(acc_sh.at[pl.ds(NS * RPT, TAIL)],
                        out_hbm.at[c, pl.ds(NS * RPT, TAIL)])


_R = 2000


def _tc_init_body(fm_ref, w1t_ref, bias_ref, w2t_ref, lin1b_ref, p_ref):
    lin1b = (jnp.dot(fm_ref[...], w1t_ref[...],
                     preferred_element_type=jnp.float32) + bias_ref[...])
    lin1b_ref[...] = lin1b
    mu = jnp.maximum(lin1b, 0.0)
    p_ref[...] = jnp.dot(mu, w2t_ref[...], preferred_element_type=jnp.float32)


def _tc_step_body(lin1b_ref, s_ref, w2t_ref, p_ref):
    mu = jnp.maximum(lin1b_ref[...] + s_ref[0] + s_ref[1], 0.0)
    p_ref[...] = jnp.dot(mu, w2t_ref[...], preferred_element_type=jnp.float32)


def _tc_final_body(lin1b_ref, s_ref, out_ref):
    mu = jnp.maximum(lin1b_ref[...] + s_ref[0] + s_ref[1], 0.0)
    part = jnp.sum(mu, axis=0, keepdims=True)

    @pl.when(pl.program_id(0) == 0)
    def _():
        out_ref[...] = jnp.zeros_like(out_ref)

    out_ref[...] += part


def _tc_init(fm, w1t, bias, w2t):
    return pl.pallas_call(
        _tc_init_body,
        grid=(NNODES // _R,),
        in_specs=[
            pl.BlockSpec((_R, NFEAT), lambda i: (i, 0)),
            pl.BlockSpec((NFEAT, ODIM), lambda i: (0, 0)),
            pl.BlockSpec((1, ODIM), lambda i: (0, 0)),
            pl.BlockSpec((ODIM, ODIM), lambda i: (0, 0)),
        ],
        out_specs=[
            pl.BlockSpec((_R, ODIM), lambda i: (i, 0)),
            pl.BlockSpec((_R, ODIM), lambda i: (i, 0)),
        ],
        out_shape=[
            jax.ShapeDtypeStruct((NNODES, ODIM), jnp.float32),
            jax.ShapeDtypeStruct((NNODES, ODIM), jnp.float32),
        ],
    )(fm, w1t, bias, w2t)


def _tc_step(lin1b, s, w2t):
    return pl.pallas_call(
        _tc_step_body,
        grid=(NNODES // _R,),
        in_specs=[
            pl.BlockSpec((_R, ODIM), lambda i: (i, 0)),
            pl.BlockSpec((NC, _R, ODIM), lambda i: (0, i, 0)),
            pl.BlockSpec((ODIM, ODIM), lambda i: (0, 0)),
        ],
        out_specs=pl.BlockSpec((_R, ODIM), lambda i: (i, 0)),
        out_shape=jax.ShapeDtypeStruct((NNODES, ODIM), jnp.float32),
    )(lin1b, s, w2t)


def _tc_final(lin1b, s):
    out = pl.pallas_call(
        _tc_final_body,
        grid=(NNODES // _R,),
        in_specs=[
            pl.BlockSpec((_R, ODIM), lambda i: (i, 0)),
            pl.BlockSpec((NC, _R, ODIM), lambda i: (0, i, 0)),
        ],
        out_specs=pl.BlockSpec((1, ODIM), lambda i: (0, 0)),
        out_shape=jax.ShapeDtypeStruct((1, ODIM), jnp.float32),
    )(lin1b, s)
    return out.reshape(ODIM)


def kernel(feature_matrix, edge_index, W1, b1, W2, b2):
    src = edge_index[0].reshape(NW, EPT)
    dst = edge_index[1].reshape(NW, EPT // 80, 80)
    w1t = W1.T
    w2t = W2.T
    bias = (b1 + b2).reshape(1, ODIM)
    zeros = jnp.zeros((RPT, ODIM), jnp.float32)

    lin1b, p = _tc_init(feature_matrix, w1t, bias, w2t)
    for _ in range(T - 2):
        s = _sc_spmm(p, src, dst, zeros)
        p = _tc_step(lin1b, s, w2t)
    s = _sc_spmm(p, src, dst, zeros)
    return _tc_final(lin1b, s)


# speedup vs baseline: 1.0056x; 1.0056x over previous
import functools

import jax
import jax.numpy as jnp
from jax import lax
from jax.experimental import pallas as pl
from jax.experimental.pallas import tpu as pltpu
from jax.experimental.pallas import tpu_sc as plsc

NNODES = 10000
NFEAT = 128
ODIM = 128
NEDGES = 320000
T = 10

NC = 2
NS = 16
NW = NC * NS
EPT = NEDGES // NW
BLK = 40
NBLK = EPT // BLK
RPT = 624
TAIL = NNODES - NS * RPT

_SC_MESH = plsc.VectorSubcoreMesh(core_axis_name="c", subcore_axis_name="s")


@functools.partial(
    pl.kernel,
    out_type=jax.ShapeDtypeStruct((NC, NNODES, ODIM), jnp.float32),
    mesh=_SC_MESH,
    scratch_types=[
        pltpu.VMEM((EPT,), jnp.int32),
        pltpu.VMEM((EPT // 80, 80), jnp.int32),
        pltpu.VMEM((4 * BLK, ODIM), jnp.float32),
        pltpu.VMEM_SHARED((NNODES, ODIM), jnp.float32),
        pltpu.SemaphoreType.DMA,
        pltpu.SemaphoreType.DMA,
        pltpu.SemaphoreType.DMA,
        pltpu.SemaphoreType.DMA,
        pltpu.SemaphoreType.DMA,
        pltpu.SemaphoreType.DMA,
    ],
)
def _sc_spmm(p_hbm, src_hbm, dst_hbm, zeros_hbm, out_hbm,
             src_v, dst_v, rows_v, acc_sh, sem0, sem1, sem2, sem3,
             isemA, isemB):
    c = lax.axis_index("c")
    s = lax.axis_index("s")
    wid = s * NC + c
    pltpu.async_copy(src_hbm.at[wid], src_v, isemA)
    pltpu.async_copy(dst_hbm.at[wid], dst_v, isemB)
    pltpu.make_async_copy(src_hbm.at[wid], src_v, isemA).wait()

    sems = (sem0, sem1, sem2, sem3)

    def gather(j, b, sem):
        jc = jnp.minimum(j, NBLK - 1)
        return pltpu.async_copy(
            p_hbm.at[src_v.at[pl.ds(jc * BLK, BLK)]],
            rows_v.at[pl.ds(b * BLK, BLK)], sem)

    def gwait(b, sem):
        pltpu.make_async_copy(
            p_hbm.at[src_v.at[pl.ds(0, BLK)]],
            rows_v.at[pl.ds(b * BLK, BLK)], sem).wait()

    def scatter_pair(p, half):
        off = half * 2 * BLK
        pltpu.sync_copy(rows_v.at[pl.ds(off, 2 * BLK)],
                        acc_sh.at[dst_v.at[p]], add=True)

    gather(0, 0, sems[0])
    gather(1, 1, sems[1])
    gather(2, 2, sems[2])
    gather(3, 3, sems[3])
    pltpu.sync_copy(zeros_hbm, acc_sh.at[pl.ds(s * RPT, RPT)])

    @pl.when(s == 0)
    def _():
        pltpu.sync_copy(zeros_hbm.at[pl.ds(0, TAIL)],
                        acc_sh.at[pl.ds(NS * RPT, TAIL)])

    pltpu.make_async_copy(dst_hbm.at[wid], dst_v, isemB).wait()
    plsc.subcore_barrier()

    def body(i, carry):
        p = 2 * i
        j = 2 * p
        gwait(0, sems[0])
        gwait(1, sems[1])
        scatter_pair(p, 0)
        gather(j + 4, 0, sems[0])
        gather(j + 5, 1, sems[1])
        gwait(2, sems[2])
        gwait(3, sems[3])
        scatter_pair(p + 1, 1)
        gather(j + 6, 2, sems[2])
        gather(j + 7, 3, sems[3])
        return carry

    lax.fori_loop(0, (NBLK // 2) // 2, body, 0)
    gwait(0, sems[0])
    gwait(1, sems[1])
    scatter_pair((NBLK // 2) - 1, 0)
    gwait(2, sems[2])
    gwait(3, sems[3])
    plsc.subcore_barrier()

    pltpu.sync_copy(acc_sh.at[pl.ds(s * RPT, RPT)],
                    out_hbm.at[c, pl.ds(s * RPT, RPT)])

    @pl.when(s == 0)
    def _():
        pltpu.sync_copy(acc_sh.at[pl.ds(NS * RPT, TAIL)],
                        out_hbm.at[c, pl.ds(NS * RPT, TAIL)])


_R = 10000


def _tc_init_body(fm_ref, w1t_ref, bias_ref, w2t_ref, lin1b_ref, p_ref):
    lin1b = (jnp.dot(fm_ref[...], w1t_ref[...],
                     preferred_element_type=jnp.float32) + bias_ref[...])
    lin1b_ref[...] = lin1b
    mu = jnp.maximum(lin1b, 0.0)
    p_ref[...] = jnp.dot(mu, w2t_ref[...], preferred_element_type=jnp.float32)


def _tc_step_body(lin1b_ref, s_ref, w2t_ref, p_ref):
    mu = jnp.maximum(lin1b_ref[...] + s_ref[0] + s_ref[1], 0.0)
    p_ref[...] = jnp.dot(mu, w2t_ref[...], preferred_element_type=jnp.float32)


def _tc_final_body(lin1b_ref, s_ref, out_ref):
    mu = jnp.maximum(lin1b_ref[...] + s_ref[0] + s_ref[1], 0.0)
    part = jnp.sum(mu, axis=0, keepdims=True)

    @pl.when(pl.program_id(0) == 0)
    def _():
        out_ref[...] = jnp.zeros_like(out_ref)

    out_ref[...] += part


def _tc_init(fm, w1t, bias, w2t):
    return pl.pallas_call(
        _tc_init_body,
        grid=(NNODES // _R,),
        in_specs=[
            pl.BlockSpec((_R, NFEAT), lambda i: (i, 0)),
            pl.BlockSpec((NFEAT, ODIM), lambda i: (0, 0)),
            pl.BlockSpec((1, ODIM), lambda i: (0, 0)),
            pl.BlockSpec((ODIM, ODIM), lambda i: (0, 0)),
        ],
        out_specs=[
            pl.BlockSpec((_R, ODIM), lambda i: (i, 0)),
            pl.BlockSpec((_R, ODIM), lambda i: (i, 0)),
        ],
        out_shape=[
            jax.ShapeDtypeStruct((NNODES, ODIM), jnp.float32),
            jax.ShapeDtypeStruct((NNODES, ODIM), jnp.float32),
        ],
    )(fm, w1t, bias, w2t)


def _tc_step(lin1b, s, w2t):
    return pl.pallas_call(
        _tc_step_body,
        grid=(NNODES // _R,),
        in_specs=[
            pl.BlockSpec((_R, ODIM), lambda i: (i, 0)),
            pl.BlockSpec((NC, _R, ODIM), lambda i: (0, i, 0)),
            pl.BlockSpec((ODIM, ODIM), lambda i: (0, 0)),
        ],
        out_specs=pl.BlockSpec((_R, ODIM), lambda i: (i, 0)),
        out_shape=jax.ShapeDtypeStruct((NNODES, ODIM), jnp.float32),
    )(lin1b, s, w2t)


def _tc_final(lin1b, s):
    out = pl.pallas_call(
        _tc_final_body,
        grid=(NNODES // _R,),
        in_specs=[
            pl.BlockSpec((_R, ODIM), lambda i: (i, 0)),
            pl.BlockSpec((NC, _R, ODIM), lambda i: (0, i, 0)),
        ],
        out_specs=pl.BlockSpec((1, ODIM), lambda i: (0, 0)),
        out_shape=jax.ShapeDtypeStruct((1, ODIM), jnp.float32),
    )(lin1b, s)
    return out.reshape(ODIM)


def kernel(feature_matrix, edge_index, W1, b1, W2, b2):
    src = edge_index[0].reshape(NW, EPT)
    dst = edge_index[1].reshape(NW, EPT // 80, 80)
    w1t = W1.T
    w2t = W2.T
    bias = (b1 + b2).reshape(1, ODIM)
    zeros = jnp.zeros((RPT, ODIM), jnp.float32)

    lin1b, p = _tc_init(feature_matrix, w1t, bias, w2t)
    for _ in range(T - 2):
        s = _sc_spmm(p, src, dst, zeros)
        p = _tc_step(lin1b, s, w2t)
    s = _sc_spmm(p, src, dst, zeros)
    return _tc_final(lin1b, s)
